# baseline (speedup 1.0000x reference)
import functools

import jax
import jax.numpy as jnp
import numpy as np
from jax import lax
from jax.experimental import pallas as pl
from jax.experimental.pallas import tpu as pltpu
from jax.experimental.pallas import tpu_sc as plsc

RATIO = 2

_NC = 2
_NS = 16

_IDX_CACHE = {}


def _sampled_idx(nums, index_num):
    if nums not in _IDX_CACHE:
        def _compute():
            perm = jax.random.permutation(jax.random.key(42), nums)
            return perm[:index_num].astype(jnp.int32)

        cpu = jax.local_devices(backend="cpu")[0]
        with jax.ensure_compile_time_eval(), jax.default_device(cpu):
            _IDX_CACHE[nums] = np.asarray(jax.jit(_compute)())
    return _IDX_CACHE[nums]


@functools.lru_cache(maxsize=None)
def _make_sampler(nums, c, index_num):
    seg = nums // _NS
    och = index_num // _NS
    rpc = c // _NC
    mesh = plsc.VectorSubcoreMesh(core_axis_name="c", subcore_axis_name="s")

    @functools.partial(
        pl.kernel,
        mesh=mesh,
        out_type=jax.ShapeDtypeStruct((c * index_num,), jnp.float32),
        scratch_types=[
            pltpu.VMEM_SHARED((nums,), jnp.float32),
            pltpu.VMEM_SHARED((nums,), jnp.float32),
            pltpu.VMEM((och,), jnp.int32),
            pltpu.VMEM((och,), jnp.float32),
            pltpu.VMEM((och,), jnp.float32),
            pltpu.SemaphoreType.DMA,
            pltpu.SemaphoreType.DMA,
            pltpu.SemaphoreType.DMA,
            pltpu.SemaphoreType.DMA,
            pltpu.SemaphoreType.DMA,
        ],
    )
    def sample_kernel(xf_hbm, idx_hbm, out_hbm, row_sh0, row_sh1, idx_v,
                      out_v0, out_v1, gsem, ssem0, ssem1, osem0, osem1):
        cid = lax.axis_index("c")
        sid = lax.axis_index("s")
        row_sh = (row_sh0, row_sh1)
        out_v = (out_v0, out_v1)
        ssem = (ssem0, ssem1)
        osem = (osem0, osem1)
        pltpu.sync_copy(idx_hbm.at[pl.ds(sid * och, och)], idx_v)

        def stage(i, b):
            r = cid * rpc + i
            return pltpu.async_copy(
                xf_hbm.at[pl.ds(r * nums + sid * seg, seg)],
                row_sh[b].at[pl.ds(sid * seg, seg)],
                ssem[b],
            )

        stage_h = [stage(0, 0), None]
        out_h = [None, None]
        for i in range(rpc):
            b = i % 2
            if i + 1 < rpc:
                stage_h[1 - b] = stage(i + 1, 1 - b)
            stage_h[b].wait()
            plsc.subcore_barrier()
            if out_h[b] is not None:
                out_h[b].wait()
            pltpu.async_copy(row_sh[b].at[idx_v], out_v[b], gsem).wait()
            r = cid * rpc + i
            out_h[b] = pltpu.async_copy(
                out_v[b],
                out_hbm.at[pl.ds(r * index_num + sid * och, och)],
                osem[b],
            )
            plsc.subcore_barrier()
        out_h[0].wait()
        out_h[1].wait()

    return sample_kernel


def kernel(x):
    b, c, nums = x.shape
    index_num = nums // RATIO
    idx = jnp.asarray(_sampled_idx(nums, index_num))
    xf = x.reshape(-1)
    out = _make_sampler(nums, c, index_num)(xf, idx)
    return out.reshape(1, c, index_num)


# speedup vs baseline: 2.4793x; 2.4793x over previous
import functools

import jax
import jax.numpy as jnp
import numpy as np
from jax import lax
from jax.experimental import pallas as pl
from jax.experimental.pallas import tpu as pltpu
from jax.experimental.pallas import tpu_sc as plsc

RATIO = 2

_NC = 2
_NS = 16

_IDX_CACHE = {}


def _sampled_idx(nums, index_num):
    if nums not in _IDX_CACHE:
        def _compute():
            perm = jax.random.permutation(jax.random.key(42), nums)
            return perm[:index_num].astype(jnp.int32)

        cpu = jax.local_devices(backend="cpu")[0]
        with jax.ensure_compile_time_eval(), jax.default_device(cpu):
            _IDX_CACHE[nums] = np.asarray(jax.jit(_compute)())
    return _IDX_CACHE[nums]


@functools.lru_cache(maxsize=None)
def _make_sampler(nums, c, index_num):
    seg = nums // _NS
    och = index_num // _NS
    rpc = c // _NC
    mesh = plsc.VectorSubcoreMesh(core_axis_name="c", subcore_axis_name="s")

    @functools.partial(
        pl.kernel,
        mesh=mesh,
        out_type=jax.ShapeDtypeStruct((c * index_num,), jnp.float32),
        scratch_types=[
            pltpu.VMEM_SHARED((nums,), jnp.float32),
            pltpu.VMEM_SHARED((nums,), jnp.float32),
            pltpu.VMEM((och,), jnp.int32),
            pltpu.VMEM((och,), jnp.float32),
            pltpu.VMEM((och,), jnp.float32),
            pltpu.SemaphoreType.DMA,
            pltpu.SemaphoreType.DMA,
            pltpu.SemaphoreType.DMA,
            pltpu.SemaphoreType.DMA,
            pltpu.SemaphoreType.DMA,
        ],
    )
    def sample_kernel(xf_hbm, idx_hbm, out_hbm, row_sh0, row_sh1, idx_v,
                      out_v0, out_v1, gsem, ssem0, ssem1, osem0, osem1):
        cid = lax.axis_index("c")
        sid = lax.axis_index("s")
        row_sh = (row_sh0, row_sh1)
        out_v = (out_v0, out_v1)
        ssem = (ssem0, ssem1)
        osem = (osem0, osem1)
        pltpu.sync_copy(idx_hbm.at[pl.ds(sid * och, och)], idx_v)

        def stage(i, b):
            r = cid * rpc + i
            return pltpu.async_copy(
                xf_hbm.at[0, r, pl.ds(sid * seg, seg)],
                row_sh[b].at[pl.ds(sid * seg, seg)],
                ssem[b],
            )

        stage_h = [stage(0, 0), None]
        out_h = [None, None]
        for i in range(rpc):
            b = i % 2
            if i + 1 < rpc:
                stage_h[1 - b] = stage(i + 1, 1 - b)
            stage_h[b].wait()
            plsc.subcore_barrier()
            if out_h[b] is not None:
                out_h[b].wait()
            pltpu.async_copy(row_sh[b].at[idx_v], out_v[b], gsem).wait()
            r = cid * rpc + i
            out_h[b] = pltpu.async_copy(
                out_v[b],
                out_hbm.at[pl.ds(r * index_num + sid * och, och)],
                osem[b],
            )
            plsc.subcore_barrier()
        out_h[0].wait()
        out_h[1].wait()

    return sample_kernel


def kernel(x):
    b, c, nums = x.shape
    index_num = nums // RATIO
    idx = jnp.asarray(_sampled_idx(nums, index_num))
    out = _make_sampler(nums, c, index_num)(x, idx)
    return out.reshape(1, c, index_num)


# speedup vs baseline: 2.5011x; 1.0088x over previous
import functools

import jax
import jax.numpy as jnp
import numpy as np
from jax import lax
from jax.experimental import pallas as pl
from jax.experimental.pallas import tpu as pltpu
from jax.experimental.pallas import tpu_sc as plsc

RATIO = 2

_NC = 2
_NS = 16

_IDX_CACHE = {}


def _sampled_idx(nums, index_num):
    if nums not in _IDX_CACHE:
        def _compute():
            perm = jax.random.permutation(jax.random.key(42), nums)
            return perm[:index_num].astype(jnp.int32)

        cpu = jax.local_devices(backend="cpu")[0]
        with jax.ensure_compile_time_eval(), jax.default_device(cpu):
            _IDX_CACHE[nums] = np.asarray(jax.jit(_compute)())
    return _IDX_CACHE[nums]


@functools.lru_cache(maxsize=None)
def _make_sampler(nums, c, index_num):
    seg = nums // _NS
    och = index_num // _NS
    rpc = c // _NC
    mesh = plsc.VectorSubcoreMesh(core_axis_name="c", subcore_axis_name="s")

    @functools.partial(
        pl.kernel,
        mesh=mesh,
        out_type=jax.ShapeDtypeStruct((c * index_num,), jnp.float32),
        scratch_types=[
            pltpu.VMEM_SHARED((nums,), jnp.float32),
            pltpu.VMEM_SHARED((nums,), jnp.float32),
            pltpu.VMEM((och,), jnp.int32),
            pltpu.VMEM((och,), jnp.float32),
            pltpu.VMEM((och,), jnp.float32),
            pltpu.SemaphoreType.DMA,
            pltpu.SemaphoreType.DMA,
            pltpu.SemaphoreType.DMA,
            pltpu.SemaphoreType.DMA,
            pltpu.SemaphoreType.DMA,
        ],
    )
    def sample_kernel(xf_hbm, idx_hbm, out_hbm, row_sh0, row_sh1, idx_v,
                      out_v0, out_v1, gsem, ssem0, ssem1, osem0, osem1):
        cid = lax.axis_index("c")
        sid = lax.axis_index("s")
        row_sh = (row_sh0, row_sh1)
        out_v = (out_v0, out_v1)
        ssem = (ssem0, ssem1)
        osem = (osem0, osem1)
        pltpu.sync_copy(idx_hbm.at[pl.ds(sid * och, och)], idx_v)

        def stage(i, b):
            r = cid * rpc + i
            return pltpu.async_copy(
                xf_hbm.at[0, r, pl.ds(sid * seg, seg)],
                row_sh[b].at[pl.ds(sid * seg, seg)],
                ssem[b],
            )

        stage_h = [stage(0, 0), None]
        out_h = [None, None]
        for i in range(rpc):
            b = i % 2
            if i + 1 < rpc:
                stage_h[1 - b] = stage(i + 1, 1 - b)
            stage_h[b].wait()
            plsc.subcore_barrier()
            if out_h[b] is not None:
                out_h[b].wait()
            gq = och // 4
            ghs = [
                pltpu.async_copy(
                    row_sh[b].at[idx_v.at[pl.ds(j * gq, gq)]],
                    out_v[b].at[pl.ds(j * gq, gq)],
                    gsem,
                )
                for j in range(4)
            ]
            for gh in ghs:
                gh.wait()
            r = cid * rpc + i
            out_h[b] = pltpu.async_copy(
                out_v[b],
                out_hbm.at[pl.ds(r * index_num + sid * och, och)],
                osem[b],
            )
            plsc.subcore_barrier()
        out_h[0].wait()
        out_h[1].wait()

    return sample_kernel


def kernel(x):
    b, c, nums = x.shape
    index_num = nums // RATIO
    idx = jnp.asarray(_sampled_idx(nums, index_num))
    out = _make_sampler(nums, c, index_num)(x, idx)
    return out.reshape(1, c, index_num)


# speedup vs baseline: 3.0797x; 1.2314x over previous
import functools

import jax
import jax.numpy as jnp
import numpy as np
from jax import lax
from jax.experimental import pallas as pl
from jax.experimental.pallas import tpu as pltpu
from jax.experimental.pallas import tpu_sc as plsc

RATIO = 2

_NC = 2
_NS = 16

_IDX_CACHE = {}


def _sampled_idx(nums, index_num):
    if nums not in _IDX_CACHE:
        def _compute():
            perm = jax.random.permutation(jax.random.key(42), nums)
            return perm[:index_num].astype(jnp.int32)

        cpu = jax.local_devices(backend="cpu")[0]
        with jax.ensure_compile_time_eval(), jax.default_device(cpu):
            _IDX_CACHE[nums] = np.asarray(jax.jit(_compute)())
    return _IDX_CACHE[nums]


@functools.lru_cache(maxsize=None)
def _make_sampler(nums, c, index_num):
    seg = nums // _NS
    och = index_num // _NS
    rpc = c // _NC
    mesh = plsc.VectorSubcoreMesh(core_axis_name="c", subcore_axis_name="s")

    @functools.partial(
        pl.kernel,
        mesh=mesh,
        out_type=jax.ShapeDtypeStruct((c, index_num), jnp.float32),
        scratch_types=[
            pltpu.VMEM_SHARED((nums,), jnp.float32),
            pltpu.VMEM_SHARED((nums,), jnp.float32),
            pltpu.VMEM((och,), jnp.int32),
            pltpu.VMEM((och,), jnp.float32),
            pltpu.VMEM((och,), jnp.float32),
            pltpu.SemaphoreType.DMA,
            pltpu.SemaphoreType.DMA,
            pltpu.SemaphoreType.DMA,
            pltpu.SemaphoreType.DMA,
            pltpu.SemaphoreType.DMA,
        ],
    )
    def sample_kernel(xf_hbm, idx_hbm, out_hbm, row_sh0, row_sh1, idx_v,
                      out_v0, out_v1, gsem, ssem0, ssem1, osem0, osem1):
        cid = lax.axis_index("c")
        sid = lax.axis_index("s")
        row_sh = (row_sh0, row_sh1)
        out_v = (out_v0, out_v1)
        ssem = (ssem0, ssem1)
        osem = (osem0, osem1)
        pltpu.sync_copy(idx_hbm.at[pl.ds(sid * och, och)], idx_v)

        def stage(i, b):
            r = cid * rpc + i
            return pltpu.async_copy(
                xf_hbm.at[0, r, pl.ds(sid * seg, seg)],
                row_sh[b].at[pl.ds(sid * seg, seg)],
                ssem[b],
            )

        stage_h = [stage(0, 0), None]
        out_h = [None, None]
        for i in range(rpc):
            b = i % 2
            if i + 1 < rpc:
                stage_h[1 - b] = stage(i + 1, 1 - b)
            stage_h[b].wait()
            plsc.subcore_barrier()
            if out_h[b] is not None:
                out_h[b].wait()
            gq = och // 4
            ghs = [
                pltpu.async_copy(
                    row_sh[b].at[idx_v.at[pl.ds(j * gq, gq)]],
                    out_v[b].at[pl.ds(j * gq, gq)],
                    gsem,
                )
                for j in range(4)
            ]
            for gh in ghs:
                gh.wait()
            r = cid * rpc + i
            out_h[b] = pltpu.async_copy(
                out_v[b],
                out_hbm.at[r, pl.ds(sid * och, och)],
                osem[b],
            )
            plsc.subcore_barrier()
        out_h[0].wait()
        out_h[1].wait()

    return sample_kernel


def kernel(x):
    b, c, nums = x.shape
    index_num = nums // RATIO
    idx = jnp.asarray(_sampled_idx(nums, index_num))
    out = _make_sampler(nums, c, index_num)(x, idx)
    return out.reshape(1, c, index_num)
